# Initial kernel scaffold; baseline (speedup 1.0000x reference)
import jax
import jax.numpy as jnp
from jax.experimental import pallas as pl


def kernel(x, edge_index, batch, conv0_W1, conv0_b1, conv0_W2, conv0_b2, conv1_W1, conv1_b1, conv1_W2, conv1_b2, conv2_W1, conv2_b1, conv2_W2, conv2_b2, ln0_g, ln0_b, ln1_g, ln1_b, post_W1, post_b1, post_W2, post_b2):
    raise NotImplementedError("write your pallas kernel here")


# speedup vs baseline: 2.9456x; 2.9456x over previous
import functools

import jax
import jax.numpy as jnp
from jax import lax
from jax.experimental import pallas as pl
from jax.experimental.pallas import tpu as pltpu
from jax.experimental.pallas import tpu_sc as plsc

_N = 10000
_E = 320000
_D = 128
_B = 16
_DOUT = 64

_NC = 2
_NS = 16
_NW = _NC * _NS
_K = 128
_NCH = 80
_EPW = _NCH * _K
_EPAD = _EPW * _NW
_NP = 10240
_RPS = _NP // _NS


def _sc_aggregate(h, eidx):
    mesh = plsc.VectorSubcoreMesh(core_axis_name="c", subcore_axis_name="s")

    @functools.partial(
        pl.kernel,
        mesh=mesh,
        out_type=jax.ShapeDtypeStruct((_NC, _NP, _D), jnp.float32),
        scratch_types=[
            pltpu.VMEM((2, _K), jnp.int32),
            pltpu.VMEM((2, _K), jnp.int32),
            pltpu.VMEM((_K, _D), jnp.float32),
            pltpu.VMEM((_K, _D), jnp.float32),
            pltpu.VMEM_SHARED((_NP, _D), jnp.float32),
            pltpu.SemaphoreType.DMA,
            pltpu.SemaphoreType.DMA,
        ],
    )
    def agg_kernel(h_hbm, eidx_hbm, out_hbm,
                   idx_a, idx_b, rows_a, rows_b, agg_sh, sem_a, sem_b):
        c = lax.axis_index("c")
        s = lax.axis_index("s")

        @pl.loop(0, _K)
        def _(r):
            for j in range(_D // 16):
                rows_a[r, pl.ds(j * 16, 16)] = jnp.zeros((16,), jnp.float32)

        @pl.loop(0, _RPS // _K)
        def _(k):
            pltpu.sync_copy(rows_a, agg_sh.at[pl.ds(s * _RPS + k * _K, _K)])

        plsc.subcore_barrier()

        pltpu.sync_copy(eidx_hbm.at[c, s, 0], idx_a)
        cp0 = pltpu.async_copy(h_hbm.at[idx_a.at[0]], rows_a, sem_a)
        pltpu.sync_copy(eidx_hbm.at[c, s, 1], idx_b)
        cp0.wait()

        @pl.loop(0, _NCH - 1)
        def _(j):
            @pl.when(j % 2 == 0)
            def _():
                cp = pltpu.async_copy(h_hbm.at[idx_b.at[0]], rows_b, sem_b)
                pltpu.sync_copy(rows_a, agg_sh.at[idx_a.at[1]], add=True)

                @pl.when(j + 2 < _NCH)
                def _():
                    pltpu.sync_copy(eidx_hbm.at[c, s, j + 2], idx_a)

                cp.wait()

            @pl.when(j % 2 == 1)
            def _():
                cp = pltpu.async_copy(h_hbm.at[idx_a.at[0]], rows_a, sem_a)
                pltpu.sync_copy(rows_b, agg_sh.at[idx_b.at[1]], add=True)

                @pl.when(j + 2 < _NCH)
                def _():
                    pltpu.sync_copy(eidx_hbm.at[c, s, j + 2], idx_b)

                cp.wait()

        pltpu.sync_copy(rows_b, agg_sh.at[idx_b.at[1]], add=True)

        plsc.subcore_barrier()

        pltpu.sync_copy(agg_sh.at[pl.ds(s * _RPS, _RPS)],
                        out_hbm.at[c, pl.ds(s * _RPS, _RPS)])

    return agg_kernel(h, eidx)[:, :_N]


_R = 1000
_G = _N // _R


def _tc_layer(h, p, W1, b1, W2, b2, g, bb):
    def body(h_ref, p0_ref, p1_ref, W1_ref, b1_ref, W2_ref, b2_ref,
             g_ref, bb_ref, out_ref):
        z = h_ref[...] + p0_ref[...] + p1_ref[...]
        a = jnp.maximum(
            jnp.dot(z, W1_ref[...], preferred_element_type=jnp.float32)
            + b1_ref[...], 0.0)
        pre = (jnp.dot(a, W2_ref[...], preferred_element_type=jnp.float32)
               + b2_ref[...])
        r = jnp.maximum(pre, 0.0)
        mu = jnp.mean(r, axis=1, keepdims=True)
        var = jnp.mean((r - mu) ** 2, axis=1, keepdims=True)
        out_ref[...] = (r - mu) * lax.rsqrt(var + 1e-5) * g_ref[...] + bb_ref[...]

    row = pl.BlockSpec((_R, _D), lambda i: (i, 0))
    full = pl.BlockSpec((_D, _D), lambda i: (0, 0))
    vec = pl.BlockSpec((1, _D), lambda i: (0, 0))
    return pl.pallas_call(
        body,
        grid=(_G,),
        in_specs=[row, row, row, full, vec, full, vec, vec, vec],
        out_specs=row,
        out_shape=jax.ShapeDtypeStruct((_N, _D), jnp.float32),
    )(h, p[0], p[1], W1, b1.reshape(1, _D), W2, b2.reshape(1, _D),
      g.reshape(1, _D), bb.reshape(1, _D))


def _tc_final(h, p, W1, b1, W2, b2, batchf, pW1, pb1, pW2p, pb2p):
    def body(h_ref, p0_ref, p1_ref, W1_ref, b1_ref, W2_ref, b2_ref,
             bf_ref, pW1_ref, pb1_ref, pW2_ref, pb2_ref,
             emb_ref, logp_ref, sums_ref, cnts_ref):
        i = pl.program_id(0)

        @pl.when(i == 0)
        def _():
            sums_ref[...] = jnp.zeros_like(sums_ref)
            cnts_ref[...] = jnp.zeros_like(cnts_ref)

        z = h_ref[...] + p0_ref[...] + p1_ref[...]
        a = jnp.maximum(
            jnp.dot(z, W1_ref[...], preferred_element_type=jnp.float32)
            + b1_ref[...], 0.0)
        emb = (jnp.dot(a, W2_ref[...], preferred_element_type=jnp.float32)
               + b2_ref[...])
        emb_ref[...] = emb
        r = jnp.maximum(emb, 0.0)

        col = lax.broadcasted_iota(jnp.int32, (_R, _D), 1).astype(jnp.float32)
        oh = jnp.where(bf_ref[...] == col, 1.0, 0.0)
        sums_ref[...] += lax.dot_general(
            oh, r, (((0,), (0,)), ((), ())),
            preferred_element_type=jnp.float32)
        cnts_ref[...] += lax.dot_general(
            oh, jnp.ones((_R, _D), jnp.float32), (((0,), (0,)), ((), ())),
            preferred_element_type=jnp.float32)

        @pl.when(i == _G - 1)
        def _():
            pooled = sums_ref[...] / jnp.maximum(cnts_ref[...], 1.0)
            o1 = (jnp.dot(pooled, pW1_ref[...],
                          preferred_element_type=jnp.float32) + pb1_ref[...])
            logits = (jnp.dot(o1, pW2_ref[...],
                              preferred_element_type=jnp.float32) + pb2_ref[...])
            cmask = lax.broadcasted_iota(jnp.int32, (_D, _D), 1) < _DOUT
            neg = jnp.float32(-1e30)
            masked = jnp.where(cmask, logits, neg)
            m = jnp.max(masked, axis=1, keepdims=True)
            e = jnp.where(cmask, jnp.exp(logits - m), 0.0)
            lse = jnp.log(jnp.sum(e, axis=1, keepdims=True))
            logp = logits - m - lse
            logp_ref[...] = lax.slice(logp, (0, 0), (_B, _D))

    row = pl.BlockSpec((_R, _D), lambda i: (i, 0))
    full = pl.BlockSpec((_D, _D), lambda i: (0, 0))
    vec = pl.BlockSpec((1, _D), lambda i: (0, 0))
    logp_spec = pl.BlockSpec((_B, _D), lambda i: (0, 0))
    return pl.pallas_call(
        body,
        grid=(_G,),
        in_specs=[row, row, row, full, vec, full, vec, row, full, vec, full,
                  vec],
        out_specs=(row, logp_spec),
        out_shape=(jax.ShapeDtypeStruct((_N, _D), jnp.float32),
                   jax.ShapeDtypeStruct((_B, _D), jnp.float32)),
        scratch_shapes=[pltpu.VMEM((_D, _D), jnp.float32),
                        pltpu.VMEM((_D, _D), jnp.float32)],
    )(h, p[0], p[1], W1, b1.reshape(1, _D), W2, b2.reshape(1, _D), batchf,
      pW1, pb1.reshape(1, _D), pW2p, pb2p)


def kernel(x, edge_index, batch,
           conv0_W1, conv0_b1, conv0_W2, conv0_b2,
           conv1_W1, conv1_b1, conv1_W2, conv1_b2,
           conv2_W1, conv2_b1, conv2_W2, conv2_b2,
           ln0_g, ln0_b, ln1_g, ln1_b,
           post_W1, post_b1, post_W2, post_b2):
    npad = _EPAD - _E
    src_p = jnp.concatenate(
        [edge_index[0], jnp.zeros((npad,), jnp.int32)])
    dst_p = jnp.concatenate(
        [edge_index[1], jnp.full((npad,), _NP - 1, jnp.int32)])
    eidx = jnp.stack([src_p.reshape(_NC, _NS, _NCH, _K),
                      dst_p.reshape(_NC, _NS, _NCH, _K)], axis=3)
    batchf = jnp.broadcast_to(
        batch.astype(jnp.float32)[:, None], (_N, _D))

    h = x
    p = _sc_aggregate(h, eidx)
    h = _tc_layer(h, p, conv0_W1, conv0_b1, conv0_W2, conv0_b2, ln0_g, ln0_b)
    p = _sc_aggregate(h, eidx)
    h = _tc_layer(h, p, conv1_W1, conv1_b1, conv1_W2, conv1_b2, ln1_g, ln1_b)
    p = _sc_aggregate(h, eidx)

    pW2p = jnp.zeros((_D, _D), jnp.float32).at[:, :_DOUT].set(post_W2)
    pb2p = jnp.zeros((1, _D), jnp.float32).at[:, :_DOUT].set(post_b2)
    emb, logp_pad = _tc_final(h, p, conv2_W1, conv2_b1, conv2_W2, conv2_b2,
                              batchf, post_W1, post_b1, pW2p, pb2p)
    return (emb, logp_pad[:, :_DOUT])


# speedup vs baseline: 3.2117x; 1.0904x over previous
import functools

import jax
import jax.numpy as jnp
from jax import lax
from jax.experimental import pallas as pl
from jax.experimental.pallas import tpu as pltpu
from jax.experimental.pallas import tpu_sc as plsc

_N = 10000
_E = 320000
_D = 128
_B = 16
_DOUT = 64

_NC = 2
_NS = 16
_NW = _NC * _NS
_K = 64
_NCH = 160
_NB = 5
_EPW = _NCH * _K
_EPAD = _EPW * _NW
_NP = 10240
_RPS = _NP // _NS


def _sc_aggregate(h, eidx):
    mesh = plsc.VectorSubcoreMesh(core_axis_name="c", subcore_axis_name="s")

    @functools.partial(
        pl.kernel,
        mesh=mesh,
        out_type=jax.ShapeDtypeStruct((_NC, _NP, _D), jnp.float32),
        scratch_types=(
            [pltpu.VMEM((2, _K), jnp.int32) for _ in range(_NB)]
            + [pltpu.VMEM((_K, _D), jnp.float32) for _ in range(_NB)]
            + [pltpu.VMEM_SHARED((_NP, _D), jnp.float32)]
            + [pltpu.SemaphoreType.DMA for _ in range(_NB)]
        ),
    )
    def agg_kernel(h_hbm, eidx_hbm, out_hbm, *scratch):
        idx = scratch[:_NB]
        rows = scratch[_NB:2 * _NB]
        agg_sh = scratch[2 * _NB]
        sem = scratch[2 * _NB + 1:]
        c = lax.axis_index("c")
        s = lax.axis_index("s")

        for b in range(2):
            @pl.loop(0, _K)
            def _(r):
                for j in range(_D // 16):
                    rows[b][r, pl.ds(j * 16, 16)] = jnp.zeros((16,), jnp.float32)

        @pl.loop(0, _RPS // (2 * _K))
        def _(k):
            pltpu.sync_copy(rows[0], agg_sh.at[pl.ds(s * _RPS + 2 * k * _K, _K)])
            pltpu.sync_copy(rows[1],
                            agg_sh.at[pl.ds(s * _RPS + (2 * k + 1) * _K, _K)])

        plsc.subcore_barrier()

        for b in range(_NB):
            pltpu.sync_copy(eidx_hbm.at[c, s, b], idx[b])
        for b in range(_NB - 1):
            pltpu.async_copy(h_hbm.at[idx[b].at[0]], rows[b], sem[b])

        @pl.loop(0, _NCH)
        def _(j):
            for b in range(_NB):
                @pl.when(j % _NB == b)
                def _(b=b):
                    pltpu.make_async_copy(h_hbm.at[idx[b].at[0]], rows[b],
                                          sem[b]).wait()
                    pltpu.sync_copy(rows[b], agg_sh.at[idx[b].at[1]], add=True)

                    @pl.when(j + _NB < _NCH)
                    def _():
                        pltpu.sync_copy(eidx_hbm.at[c, s, j + _NB], idx[b])

                    nb = (b + _NB - 1) % _NB
                    @pl.when(j + _NB - 1 < _NCH)
                    def _():
                        pltpu.async_copy(h_hbm.at[idx[nb].at[0]], rows[nb],
                                         sem[nb])

        plsc.subcore_barrier()

        pltpu.sync_copy(agg_sh.at[pl.ds(s * _RPS, _RPS)],
                        out_hbm.at[c, pl.ds(s * _RPS, _RPS)])

    return agg_kernel(h, eidx)[:, :_N]


_R = 1000
_G = _N // _R


def _tc_layer(h, p, W1, b1, W2, b2, g, bb):
    def body(h_ref, p0_ref, p1_ref, W1_ref, b1_ref, W2_ref, b2_ref,
             g_ref, bb_ref, out_ref):
        z = h_ref[...] + p0_ref[...] + p1_ref[...]
        a = jnp.maximum(
            jnp.dot(z, W1_ref[...], preferred_element_type=jnp.float32)
            + b1_ref[...], 0.0)
        pre = (jnp.dot(a, W2_ref[...], preferred_element_type=jnp.float32)
               + b2_ref[...])
        r = jnp.maximum(pre, 0.0)
        mu = jnp.mean(r, axis=1, keepdims=True)
        var = jnp.mean((r - mu) ** 2, axis=1, keepdims=True)
        out_ref[...] = (r - mu) * lax.rsqrt(var + 1e-5) * g_ref[...] + bb_ref[...]

    row = pl.BlockSpec((_R, _D), lambda i: (i, 0))
    full = pl.BlockSpec((_D, _D), lambda i: (0, 0))
    vec = pl.BlockSpec((1, _D), lambda i: (0, 0))
    return pl.pallas_call(
        body,
        grid=(_G,),
        in_specs=[row, row, row, full, vec, full, vec, vec, vec],
        out_specs=row,
        out_shape=jax.ShapeDtypeStruct((_N, _D), jnp.float32),
    )(h, p[0], p[1], W1, b1.reshape(1, _D), W2, b2.reshape(1, _D),
      g.reshape(1, _D), bb.reshape(1, _D))


def _tc_final(h, p, W1, b1, W2, b2, batchf, pW1, pb1, pW2p, pb2p):
    def body(h_ref, p0_ref, p1_ref, W1_ref, b1_ref, W2_ref, b2_ref,
             bf_ref, pW1_ref, pb1_ref, pW2_ref, pb2_ref,
             emb_ref, logp_ref, sums_ref, cnts_ref):
        i = pl.program_id(0)

        @pl.when(i == 0)
        def _():
            sums_ref[...] = jnp.zeros_like(sums_ref)
            cnts_ref[...] = jnp.zeros_like(cnts_ref)

        z = h_ref[...] + p0_ref[...] + p1_ref[...]
        a = jnp.maximum(
            jnp.dot(z, W1_ref[...], preferred_element_type=jnp.float32)
            + b1_ref[...], 0.0)
        emb = (jnp.dot(a, W2_ref[...], preferred_element_type=jnp.float32)
               + b2_ref[...])
        emb_ref[...] = emb
        r = jnp.maximum(emb, 0.0)

        col = lax.broadcasted_iota(jnp.int32, (_R, _D), 1).astype(jnp.float32)
        oh = jnp.where(bf_ref[...] == col, 1.0, 0.0)
        sums_ref[...] += lax.dot_general(
            oh, r, (((0,), (0,)), ((), ())),
            preferred_element_type=jnp.float32)
        cnts_ref[...] += lax.dot_general(
            oh, jnp.ones((_R, _D), jnp.float32), (((0,), (0,)), ((), ())),
            preferred_element_type=jnp.float32)

        @pl.when(i == _G - 1)
        def _():
            pooled = sums_ref[...] / jnp.maximum(cnts_ref[...], 1.0)
            o1 = (jnp.dot(pooled, pW1_ref[...],
                          preferred_element_type=jnp.float32) + pb1_ref[...])
            logits = (jnp.dot(o1, pW2_ref[...],
                              preferred_element_type=jnp.float32) + pb2_ref[...])
            cmask = lax.broadcasted_iota(jnp.int32, (_D, _D), 1) < _DOUT
            neg = jnp.float32(-1e30)
            masked = jnp.where(cmask, logits, neg)
            m = jnp.max(masked, axis=1, keepdims=True)
            e = jnp.where(cmask, jnp.exp(logits - m), 0.0)
            lse = jnp.log(jnp.sum(e, axis=1, keepdims=True))
            logp = logits - m - lse
            logp_ref[...] = lax.slice(logp, (0, 0), (_B, _D))

    row = pl.BlockSpec((_R, _D), lambda i: (i, 0))
    full = pl.BlockSpec((_D, _D), lambda i: (0, 0))
    vec = pl.BlockSpec((1, _D), lambda i: (0, 0))
    logp_spec = pl.BlockSpec((_B, _D), lambda i: (0, 0))
    return pl.pallas_call(
        body,
        grid=(_G,),
        in_specs=[row, row, row, full, vec, full, vec, row, full, vec, full,
                  vec],
        out_specs=(row, logp_spec),
        out_shape=(jax.ShapeDtypeStruct((_N, _D), jnp.float32),
                   jax.ShapeDtypeStruct((_B, _D), jnp.float32)),
        scratch_shapes=[pltpu.VMEM((_D, _D), jnp.float32),
                        pltpu.VMEM((_D, _D), jnp.float32)],
    )(h, p[0], p[1], W1, b1.reshape(1, _D), W2, b2.reshape(1, _D), batchf,
      pW1, pb1.reshape(1, _D), pW2p, pb2p)


def kernel(x, edge_index, batch,
           conv0_W1, conv0_b1, conv0_W2, conv0_b2,
           conv1_W1, conv1_b1, conv1_W2, conv1_b2,
           conv2_W1, conv2_b1, conv2_W2, conv2_b2,
           ln0_g, ln0_b, ln1_g, ln1_b,
           post_W1, post_b1, post_W2, post_b2):
    npad = _EPAD - _E
    src_p = jnp.concatenate(
        [edge_index[0], jnp.zeros((npad,), jnp.int32)])
    dst_p = jnp.concatenate(
        [edge_index[1], jnp.full((npad,), _NP - 1, jnp.int32)])
    eidx = jnp.stack([src_p.reshape(_NC, _NS, _NCH, _K),
                      dst_p.reshape(_NC, _NS, _NCH, _K)], axis=3)
    batchf = jnp.broadcast_to(
        batch.astype(jnp.float32)[:, None], (_N, _D))

    h = x
    p = _sc_aggregate(h, eidx)
    h = _tc_layer(h, p, conv0_W1, conv0_b1, conv0_W2, conv0_b2, ln0_g, ln0_b)
    p = _sc_aggregate(h, eidx)
    h = _tc_layer(h, p, conv1_W1, conv1_b1, conv1_W2, conv1_b2, ln1_g, ln1_b)
    p = _sc_aggregate(h, eidx)

    pW2p = jnp.zeros((_D, _D), jnp.float32).at[:, :_DOUT].set(post_W2)
    pb2p = jnp.zeros((1, _D), jnp.float32).at[:, :_DOUT].set(post_b2)
    emb, logp_pad = _tc_final(h, p, conv2_W1, conv2_b1, conv2_W2, conv2_b2,
                              batchf, post_W1, post_b1, pW2p, pb2p)
    return (emb, logp_pad[:, :_DOUT])
